# Initial kernel scaffold; baseline (speedup 1.0000x reference)
import jax
import jax.numpy as jnp
from jax.experimental import pallas as pl


def kernel(x_fire, x_weather, x_terrain, edge_index, Wf, bf, Ww, bw, Wt, bt, Wg0, bg0, Wg1, bg1, Wg2, bg2, Wo1, bo1, Wo2, bo2):
    raise NotImplementedError("write your pallas kernel here")


# speedup vs baseline: 10.6302x; 10.6302x over previous
import functools

import jax
import jax.numpy as jnp
from jax import lax
from jax.experimental import pallas as pl
from jax.experimental.pallas import tpu as pltpu
from jax.experimental.pallas import tpu_sc as plsc

N = 50000
E = 800000
H = 64

NC = 2
NS = 16
L = 16
CHUNK = 128
SLOTS = 3
GB = CHUNK * SLOTS

HALF = N // NC
GARB = 64
SH_ROWS = HALF + GARB

EP_T = ((E + NS * GB - 1) // (NS * GB)) * GB
E_PAD = NS * EP_T

ROWS_PT = 1568


def _owned_start(s):
    return jnp.minimum(s * ROWS_PT, HALF - ROWS_PT)


_MESH = plsc.VectorSubcoreMesh(core_axis_name="c", subcore_axis_name="s")
_SC_PARAMS = pltpu.CompilerParams(use_tc_tiling_on_sc=False)


def _adjust_dst(dst_v, d0, loc_v, j, base):
    for k in range(CHUNK // L):
        d = dst_v[pl.ds(d0 + k * L, L)]
        dl = d - base
        ok = (dl >= 0) & (dl < HALF)
        garb = HALF + (d & (GARB - 1))
        loc_v[j, pl.ds(k * L, L)] = jnp.where(ok, dl, garb)


def _deg_body(dstp_hbm, ones_hbm, zeros_hbm, deg_hbm,
              hist_sh, dst_v, loc_v, ones_v, ssem):
    c = lax.axis_index("c")
    s = lax.axis_index("s")
    base = c * HALF
    r0 = _owned_start(s)
    pltpu.sync_copy(zeros_hbm, hist_sh.at[pl.ds(r0, ROWS_PT)])
    pltpu.sync_copy(ones_hbm, ones_v)
    plsc.subcore_barrier()

    t0 = s * EP_T

    @pl.loop(0, EP_T // GB)
    def _(i):
        off = t0 + i * GB
        pltpu.sync_copy(dstp_hbm.at[pl.ds(off, GB)], dst_v)
        @pl.when(i > 0)
        def _():
            for j in range(SLOTS):
                pltpu.make_async_copy(ones_hbm, ones_v, ssem).wait()
        for j in range(SLOTS):
            _adjust_dst(dst_v, j * CHUNK, loc_v, j, base)
        for j in range(SLOTS):
            pltpu.async_copy(ones_v, hist_sh.at[loc_v.at[j]], ssem, add=True)

    for j in range(SLOTS):
        pltpu.make_async_copy(ones_hbm, ones_v, ssem).wait()
    plsc.subcore_barrier()
    pltpu.sync_copy(hist_sh.at[pl.ds(r0, ROWS_PT)],
                    deg_hbm.at[pl.ds(base + r0, ROWS_PT)])


_deg_kernel = functools.partial(
    pl.kernel,
    _deg_body,
    jax.ShapeDtypeStruct((N, L), jnp.float32),
    mesh=_MESH,
    compiler_params=_SC_PARAMS,
    scratch_types=[
        pltpu.VMEM_SHARED((SH_ROWS, L), jnp.float32),
        pltpu.VMEM((GB,), jnp.int32),
        pltpu.VMEM((SLOTS, CHUNK), jnp.int32),
        pltpu.VMEM((CHUNK, L), jnp.float32),
        pltpu.SemaphoreType.DMA,
    ],
)


def _agg_body(y_hbm, srcp_hbm, dstp_hbm, out_hbm,
              acc_sh, src_v, dst_v, loc_v, rows_v, gsem, ssem):
    c = lax.axis_index("c")
    s = lax.axis_index("s")
    base = c * HALF
    r0 = _owned_start(s)
    pltpu.sync_copy(y_hbm.at[pl.ds(base + r0, ROWS_PT)],
                    acc_sh.at[pl.ds(r0, ROWS_PT)])
    plsc.subcore_barrier()

    t0 = s * EP_T
    dummy = y_hbm.at[pl.ds(0, CHUNK)]

    @pl.loop(0, EP_T // GB)
    def _(i):
        off = t0 + i * GB
        pltpu.sync_copy(srcp_hbm.at[pl.ds(off, GB)], src_v)
        pltpu.sync_copy(dstp_hbm.at[pl.ds(off, GB)], dst_v)
        @pl.when(i > 0)
        def _():
            for j in range(SLOTS):
                pltpu.make_async_copy(dummy, rows_v.at[j], ssem).wait()
        for j in range(SLOTS):
            pltpu.async_copy(y_hbm.at[src_v.at[pl.ds(j * CHUNK, CHUNK)]],
                             rows_v.at[j], gsem)
        for j in range(SLOTS):
            _adjust_dst(dst_v, j * CHUNK, loc_v, j, base)
        for j in range(SLOTS):
            pltpu.make_async_copy(dummy, rows_v.at[j], gsem).wait()
        for j in range(SLOTS):
            pltpu.async_copy(rows_v.at[j], acc_sh.at[loc_v.at[j]], ssem,
                             add=True)

    for j in range(SLOTS):
        pltpu.make_async_copy(dummy, rows_v.at[j], ssem).wait()
    plsc.subcore_barrier()
    pltpu.sync_copy(acc_sh.at[pl.ds(r0, ROWS_PT)],
                    out_hbm.at[pl.ds(base + r0, ROWS_PT)])


_agg_kernel = functools.partial(
    pl.kernel,
    _agg_body,
    jax.ShapeDtypeStruct((N, H), jnp.float32),
    mesh=_MESH,
    compiler_params=_SC_PARAMS,
    scratch_types=[
        pltpu.VMEM_SHARED((SH_ROWS, H), jnp.float32),
        pltpu.VMEM((GB,), jnp.int32),
        pltpu.VMEM((GB,), jnp.int32),
        pltpu.VMEM((SLOTS, CHUNK), jnp.int32),
        pltpu.VMEM((SLOTS, CHUNK, H), jnp.float32),
        pltpu.SemaphoreType.DMA,
        pltpu.SemaphoreType.DMA,
    ],
)


BLK = 2000


def _dinv_of(deg_ref):
    return 1.0 / jnp.sqrt(deg_ref[:, 0:1] + 1.0)


def _dot(a, b, precision=None):
    return jax.lax.dot_general(a, b, (((1,), (0,)), ((), ())),
                               precision=precision,
                               preferred_element_type=jnp.float32)


def _enc_body(xf, xw, xt, deg, Wf, bf, Ww, bw, Wt, bt, oa, ob, oc):
    dinv = _dinv_of(deg)
    oa[...] = dinv * jnp.maximum(_dot(xf[...], Wf[...]) + bf[...], 0.0)
    ob[...] = dinv * jnp.maximum(_dot(xw[...], Ww[...]) + bw[...], 0.0)
    oc[...] = dinv * jnp.maximum(_dot(xt[...], Wt[...]) + bt[...], 0.0)


def _l0_body(ta, tb, tc, deg, Wg0, b, out):
    dinv = _dinv_of(deg)
    w = Wg0[...]
    z = (_dot(dinv * ta[...], w[0:H])
         + _dot(dinv * tb[...], w[H:2 * H])
         + _dot(dinv * tc[...], w[2 * H:3 * H]))
    x = jnp.maximum(z + b[...], 0.0)
    out[...] = dinv * x


def _mid_body(t_in, deg, W, b, out):
    dinv = _dinv_of(deg)
    x = jnp.maximum(_dot(dinv * t_in[...], W[...]) + b[...], 0.0)
    out[...] = dinv * x


def _head_body(t_in, deg, Wg, bg, Wo1, bo1, Wo2, bo2, out):
    dinv = _dinv_of(deg)
    x = jnp.maximum(_dot(dinv * t_in[...], Wg[...]) + bg[...], 0.0)
    h = jnp.maximum(_dot(x, Wo1[...]) + bo1[...], 0.0)
    out[...] = _dot(h, Wo2[...]) + bo2[...]


def _full(shape):
    return pl.BlockSpec(shape, lambda i: (0,) * len(shape))


def _rows(width):
    return pl.BlockSpec((BLK, width), lambda i: (i, 0))


def kernel(x_fire, x_weather, x_terrain, edge_index,
           Wf, bf, Ww, bw, Wt, bt,
           Wg0, bg0, Wg1, bg1, Wg2, bg2,
           Wo1, bo1, Wo2, bo2):
    f32 = jnp.float32
    src = edge_index[0]
    dst = edge_index[1]
    pad = E_PAD - E
    srcp = jnp.concatenate([src, jnp.zeros((pad,), jnp.int32)])
    dstp = jnp.concatenate([dst, jnp.full((pad,), N, jnp.int32)])
    ones_hbm = jnp.ones((CHUNK, L), f32)
    zeros_hbm = jnp.zeros((ROWS_PT, L), f32)

    deg16 = _deg_kernel()(dstp, ones_hbm, zeros_hbm)

    grid = (N // BLK,)
    nh = jax.ShapeDtypeStruct((N, H), f32)
    enc = pl.pallas_call(
        _enc_body,
        out_shape=(nh, nh, nh),
        grid=grid,
        in_specs=[_rows(8), _rows(12), _rows(10), _rows(L),
                  _full((8, H)), _full((1, H)),
                  _full((12, H)), _full((1, H)),
                  _full((10, H)), _full((1, H))],
        out_specs=(_rows(H), _rows(H), _rows(H)),
    )
    ua, ub, uc = enc(x_fire, x_weather, x_terrain, deg16,
                     Wf, bf.reshape(1, H), Ww, bw.reshape(1, H),
                     Wt, bt.reshape(1, H))

    ta = _agg_kernel()(ua, srcp, dstp)
    tb = _agg_kernel()(ub, srcp, dstp)
    tc = _agg_kernel()(uc, srcp, dstp)
    u1 = pl.pallas_call(
        _l0_body,
        out_shape=nh,
        grid=grid,
        in_specs=[_rows(H), _rows(H), _rows(H), _rows(L),
                  _full((3 * H, H)), _full((1, H))],
        out_specs=_rows(H),
    )(ta, tb, tc, deg16, Wg0, bg0.reshape(1, H))
    t1 = _agg_kernel()(u1, srcp, dstp)
    u2 = pl.pallas_call(
        _mid_body,
        out_shape=nh,
        grid=grid,
        in_specs=[_rows(H), _rows(L), _full((H, H)), _full((1, H))],
        out_specs=_rows(H),
    )(t1, deg16, Wg1, bg1.reshape(1, H))
    t2 = _agg_kernel()(u2, srcp, dstp)

    head = pl.pallas_call(
        _head_body,
        out_shape=jax.ShapeDtypeStruct((N, 1), f32),
        grid=grid,
        in_specs=[_rows(H), _rows(L), _full((H, H)), _full((1, H)),
                  _full((H, H // 2)), _full((1, H // 2)),
                  _full((H // 2, 1)), _full((1, 1))],
        out_specs=_rows(1),
    )
    out = head(t2, deg16, Wg2, bg2.reshape(1, H), Wo1, bo1.reshape(1, H // 2),
               Wo2, bo2.reshape(1, 1))
    return out
